# Initial kernel scaffold; baseline (speedup 1.0000x reference)
import jax
import jax.numpy as jnp
from jax.experimental import pallas as pl


def kernel(x, lens):
    raise NotImplementedError("write your pallas kernel here")


# speedup vs baseline: 1.4179x; 1.4179x over previous
import functools

import jax
import jax.numpy as jnp
from jax import lax
from jax.experimental import pallas as pl
from jax.experimental.pallas import tpu as pltpu
from jax.experimental.pallas import tpu_sc as plsc

_T, _B, _D = 2048, 16, 1024
_T2 = _T // 2
_R = _T * _B
_NC, _NS, _L = 2, 16, 16
_NW = _NC * _NS
_STRIPES = _T2 // _NW
_CHUNKS = 2 * _STRIPES


@functools.partial(
    pl.kernel,
    out_type=jax.ShapeDtypeStruct((_R, _D), jnp.float32),
    mesh=plsc.VectorSubcoreMesh(core_axis_name="c", subcore_axis_name="s"),
    scratch_types=[
        pltpu.VMEM((2, _L), jnp.int32),
        pltpu.VMEM((_L,), jnp.int32),
        pltpu.VMEM((_L, _D), jnp.float32),
        pltpu.SemaphoreType.DMA,
    ],
)
def _pool_sc(x_hbm, lens_rep_hbm, out_hbm, lens_v, idx_v, buf_v, sem):
    wid = lax.axis_index("s") * _NC + lax.axis_index("c")
    pltpu.sync_copy(lens_rep_hbm, lens_v)
    lane = lax.iota(jnp.int32, _L)
    j_vec = lane & 1
    bh_vec = lane >> 1

    @pl.loop(0, _CHUNKS)
    def _chunk(k):
        t2 = wid + (k >> 1) * _NW
        h = k & 1
        b_vec = bh_vec + 8 * h
        lens_b = lens_v[h]
        t_nat = 2 * t2 + j_vec
        masked = (lens_b >> 1) <= t2
        t_src = jnp.where(masked, jnp.maximum(t_nat, lens_b), t_nat)
        idx_v[...] = t_src * _B + b_vec
        pltpu.async_copy(x_hbm.at[idx_v], buf_v, sem).wait()
        r_base = t2 * 32 + h * 16
        pltpu.sync_copy(buf_v, out_hbm.at[pl.ds(r_base, _L)])


def kernel(x, lens):
    T, B, D = x.shape
    lens_rep = jnp.repeat(lens.astype(jnp.int32), 2).reshape(2, _L)
    out_flat = _pool_sc(x.reshape(T * B, D), lens_rep)
    out = out_flat.reshape(T // 2, B, 2 * D)
    return out, lens // 2


# speedup vs baseline: 1.6731x; 1.1800x over previous
import functools

import jax
import jax.numpy as jnp
from jax import lax
from jax.experimental import pallas as pl
from jax.experimental.pallas import tpu as pltpu
from jax.experimental.pallas import tpu_sc as plsc

_T, _B, _D = 2048, 16, 1024
_T2 = _T // 2
_R = _T * _B
_NC, _NS, _L = 2, 16, 16
_NW = _NC * _NS
_N = _T2 // _NW
_CH = 2 * _B


@functools.partial(
    pl.kernel,
    out_type=jax.ShapeDtypeStruct((_R, _D), jnp.float32),
    mesh=plsc.VectorSubcoreMesh(core_axis_name="c", subcore_axis_name="s"),
    scratch_types=[
        pltpu.VMEM((2, _L), jnp.int32),
        pltpu.VMEM((2, _CH), jnp.int32),
        pltpu.VMEM((2, _CH, _D), jnp.float32),
        pltpu.SemaphoreType.DMA,
        pltpu.SemaphoreType.DMA,
        pltpu.SemaphoreType.DMA,
        pltpu.SemaphoreType.DMA,
    ],
)
def _pool_sc(x_hbm, lens_rep_hbm, out_hbm, lens_v, idx_v, buf_v,
             gsem0, gsem1, wsem0, wsem1):
    wid = lax.axis_index("s") * _NC + lax.axis_index("c")
    pltpu.sync_copy(lens_rep_hbm, lens_v)
    lane = lax.iota(jnp.int32, _L)
    j_vec = lane & 1
    bh_vec = lane >> 1
    gsems = (gsem0, gsem1)
    wsems = (wsem0, wsem1)

    def fill_idx(k, slot):
        t2 = wid + k * _NW
        t_nat = 2 * t2 + j_vec
        for h in range(2):
            b_vec = bh_vec + 8 * h
            lens_b = lens_v[h]
            masked = (lens_b >> 1) <= t2
            t_src = jnp.where(masked, jnp.maximum(t_nat, lens_b), t_nat)
            idx_v[slot, pl.ds(_L * h, _L)] = t_src * _B + b_vec

    def gather(slot):
        return pltpu.make_async_copy(
            x_hbm.at[idx_v.at[slot]], buf_v.at[slot], gsems[slot])

    def write(k, slot):
        r_base = (wid + k * _NW) * _CH
        return pltpu.make_async_copy(
            buf_v.at[slot], out_hbm.at[pl.ds(r_base, _CH)], wsems[slot])

    fill_idx(0, 0)
    gather(0).start()

    @pl.loop(0, _N, step=2)
    def _pair(g):
        for b in range(2):
            k = g + b
            slot, nslot = b, 1 - b

            def prefetch():
                fill_idx(k + 1, nslot)

                @pl.when(k >= 1)
                def _drain():
                    write(k - 1, nslot).wait()

                gather(nslot).start()

            if b == 0:
                prefetch()
            else:
                pl.when(k + 1 < _N)(prefetch)

            gather(slot).wait()
            write(k, slot).start()

    write(_N - 2, 0).wait()
    write(_N - 1, 1).wait()


def kernel(x, lens):
    T, B, D = x.shape
    lens_rep = jnp.repeat(lens.astype(jnp.int32), 2).reshape(2, _L)
    out_flat = _pool_sc(x.reshape(T * B, D), lens_rep)
    out = out_flat.reshape(T // 2, B, 2 * D)
    return out, lens // 2


# speedup vs baseline: 3.7756x; 2.2567x over previous
import functools

import jax
import jax.numpy as jnp
from jax import lax
from jax.experimental import pallas as pl
from jax.experimental.pallas import tpu as pltpu
from jax.experimental.pallas import tpu_sc as plsc

_T, _B, _D = 2048, 16, 1024
_T2 = _T // 2
_NC, _NS, _L = 2, 16, 16
_NW = _NC * _NS
_N = _T2 // _NW


@functools.partial(
    pl.kernel,
    out_type=jax.ShapeDtypeStruct((_T2 * _B, 2 * _D), jnp.float32),
    mesh=plsc.VectorSubcoreMesh(core_axis_name="c", subcore_axis_name="s"),
    scratch_types=[
        pltpu.VMEM((_B,), jnp.int32),
        pltpu.VMEM((2, 2, _B), jnp.int32),
        pltpu.VMEM((2, _B, 2 * _D), jnp.float32),
        pltpu.SemaphoreType.DMA,
        pltpu.SemaphoreType.DMA,
        pltpu.SemaphoreType.DMA,
        pltpu.SemaphoreType.DMA,
    ],
)
def _pool_sc(x_hbm, lens_hbm, out_hbm, lens_v, idx_v, buf_v,
             gsem0, gsem1, wsem0, wsem1):
    wid = lax.axis_index("s") * _NC + lax.axis_index("c")
    pltpu.sync_copy(lens_hbm, lens_v)
    lens_b = lens_v[...]
    newlens = lens_b >> 1
    b_vec = lax.iota(jnp.int32, _L)
    gsems = (gsem0, gsem1)
    wsems = (wsem0, wsem1)

    def fill_idx(k, slot):
        t2 = wid + k * _NW
        masked = newlens <= t2
        for j in range(2):
            t_nat = 2 * t2 + j
            t_src = jnp.where(masked, jnp.maximum(t_nat, lens_b), t_nat)
            idx_v[slot, j] = t_src * _B + b_vec

    def gathers(slot):
        return (
            pltpu.make_async_copy(
                x_hbm.at[idx_v.at[slot, 0]],
                buf_v.at[slot, :, pl.ds(0, _D)], gsems[slot]),
            pltpu.make_async_copy(
                x_hbm.at[idx_v.at[slot, 1]],
                buf_v.at[slot, :, pl.ds(_D, _D)], gsems[slot]),
        )

    def write(k, slot):
        q_base = (wid + k * _NW) * _B
        return pltpu.make_async_copy(
            buf_v.at[slot], out_hbm.at[pl.ds(q_base, _B)], wsems[slot])

    def start_gathers(slot):
        for g in gathers(slot):
            g.start()

    def wait_gathers(slot):
        for g in gathers(slot):
            g.wait()

    fill_idx(0, 0)
    start_gathers(0)

    @pl.loop(0, _N, step=2)
    def _pair(g):
        for b in range(2):
            k = g + b
            slot, nslot = b, 1 - b

            def prefetch():
                fill_idx(k + 1, nslot)

                @pl.when(k >= 1)
                def _drain():
                    write(k - 1, nslot).wait()

                start_gathers(nslot)

            if b == 0:
                prefetch()
            else:
                pl.when(k + 1 < _N)(prefetch)

            wait_gathers(slot)
            write(k, slot).start()

    write(_N - 2, 0).wait()
    write(_N - 1, 1).wait()


def kernel(x, lens):
    T, B, D = x.shape
    out_flat = _pool_sc(x.reshape(T * B, D), lens.astype(jnp.int32))
    out = out_flat.reshape(T // 2, B, 2 * D)
    return out, lens // 2


# speedup vs baseline: 3.8131x; 1.0099x over previous
import functools

import jax
import jax.numpy as jnp
from jax import lax
from jax.experimental import pallas as pl
from jax.experimental.pallas import tpu as pltpu
from jax.experimental.pallas import tpu_sc as plsc

_T, _B, _D = 2048, 16, 1024
_T2 = _T // 2
_NC, _NS, _L = 2, 16, 16
_NW = _NC * _NS
_N = _T2 // _NW


@functools.partial(
    pl.kernel,
    out_type=jax.ShapeDtypeStruct((_T2 * _B, 2 * _D), jnp.float32),
    mesh=plsc.VectorSubcoreMesh(core_axis_name="c", subcore_axis_name="s"),
    scratch_types=[
        pltpu.VMEM((_B,), jnp.int32),
        pltpu.VMEM((3, 2, _B), jnp.int32),
        pltpu.VMEM((3, _B, 2 * _D), jnp.float32),
        pltpu.SemaphoreType.DMA,
        pltpu.SemaphoreType.DMA,
        pltpu.SemaphoreType.DMA,
        pltpu.SemaphoreType.DMA,
        pltpu.SemaphoreType.DMA,
        pltpu.SemaphoreType.DMA,
    ],
)
def _pool_sc(x_hbm, lens_hbm, out_hbm, lens_v, idx_v, buf_v,
             gsem0, gsem1, gsem2, wsem0, wsem1, wsem2):
    wid = lax.axis_index("s") * _NC + lax.axis_index("c")
    pltpu.sync_copy(lens_hbm, lens_v)
    lens_b = lens_v[...]
    newlens = lens_b >> 1
    b_vec = lax.iota(jnp.int32, _L)
    gsems = (gsem0, gsem1, gsem2)
    wsems = (wsem0, wsem1, wsem2)

    def fill_idx(k, slot):
        t2 = wid + k * _NW
        masked = newlens <= t2
        for j in range(2):
            t_nat = 2 * t2 + j
            t_src = jnp.where(masked, jnp.maximum(t_nat, lens_b), t_nat)
            idx_v[slot, j] = t_src * _B + b_vec

    def gathers(slot):
        return (
            pltpu.make_async_copy(
                x_hbm.at[idx_v.at[slot, 0]],
                buf_v.at[slot, :, pl.ds(0, _D)], gsems[slot]),
            pltpu.make_async_copy(
                x_hbm.at[idx_v.at[slot, 1]],
                buf_v.at[slot, :, pl.ds(_D, _D)], gsems[slot]),
        )

    def write(k, slot):
        q_base = (wid + k * _NW) * _B
        return pltpu.make_async_copy(
            buf_v.at[slot], out_hbm.at[pl.ds(q_base, _B)], wsems[slot])

    def start_gathers(slot):
        for g in gathers(slot):
            g.start()

    def wait_gathers(slot):
        for g in gathers(slot):
            g.wait()

    for k0 in range(2):
        fill_idx(k0, k0)
        start_gathers(k0)

    @pl.loop(0, _N - 2, step=3)
    def _triple(g):
        for b in range(3):
            k = g + b
            slot, nslot = b, (b + 2) % 3

            def prefetch():
                fill_idx(k + 2, nslot)

                @pl.when(k >= 1)
                def _drain():
                    write(k - 1, nslot).wait()

                start_gathers(nslot)

            prefetch()

            wait_gathers(slot)
            write(k, slot).start()

    for k in (_N - 2, _N - 1):
        wait_gathers(k % 3)
        write(k, k % 3).start()
    write(_N - 3, (_N - 3) % 3).wait()
    write(_N - 2, (_N - 2) % 3).wait()
    write(_N - 1, (_N - 1) % 3).wait()


def kernel(x, lens):
    T, B, D = x.shape
    out_flat = _pool_sc(x.reshape(T * B, D), lens.astype(jnp.int32))
    out = out_flat.reshape(T // 2, B, 2 * D)
    return out, lens // 2


# speedup vs baseline: 4.2712x; 1.1201x over previous
import functools

import jax
import jax.numpy as jnp
from jax import lax
from jax.experimental import pallas as pl
from jax.experimental.pallas import tpu as pltpu
from jax.experimental.pallas import tpu_sc as plsc

_T, _B, _D = 2048, 16, 1024
_T2 = _T // 2
_NC, _NS, _L = 2, 16, 16
_NW = _NC * _NS
_BLK = 16
_NBLK = _T2 // _BLK
_N = _NBLK * _B // _NW


@functools.partial(
    pl.kernel,
    out_type=jax.ShapeDtypeStruct((_T2, _B, 2 * _D), jnp.float32),
    mesh=plsc.VectorSubcoreMesh(core_axis_name="c", subcore_axis_name="s"),
    scratch_types=[
        pltpu.VMEM((_B, _L), jnp.int32),
        pltpu.VMEM((2, 2, _L), jnp.int32),
        pltpu.VMEM((2, _BLK, 2 * _D), jnp.float32),
        pltpu.VMEM((_BLK, 2 * _D), jnp.float32),
        pltpu.SemaphoreType.DMA,
        pltpu.SemaphoreType.DMA,
        pltpu.SemaphoreType.DMA,
        pltpu.SemaphoreType.DMA,
    ],
)
def _pool_sc(x_hbm, lens_bc_hbm, zeros_hbm, out_hbm,
             lens_v, idx_v, buf_v, zbuf_v, gsem0, gsem1, wsem0, wsem1):
    wid = lax.axis_index("s") * _NC + lax.axis_index("c")
    pltpu.sync_copy(lens_bc_hbm, lens_v)
    pltpu.sync_copy(zeros_hbm, zbuf_v)
    i_vec = lax.iota(jnp.int32, _L)
    gsems = (gsem0, gsem1)
    wsems = (wsem0, wsem1)

    def item(k):
        blk = wid + _NW * (k >> 4)
        b = k & (_B - 1)
        return blk * _BLK, b

    def item_valid(k):
        t2_0, b = item(k)
        newlens = lens_v[b] >> 1
        return t2_0 < newlens[0]

    def fill_idx(k, slot):
        t2_0, b = item(k)
        lens_b = lens_v[b]
        t2_vec = t2_0 + i_vec
        masked = (lens_b >> 1) <= t2_vec
        for j in range(2):
            t_nat = 2 * t2_vec + j
            t_src = jnp.where(masked, jnp.maximum(t_nat, lens_b), t_nat)
            idx_v[slot, j] = t_src * _B + b

    def gathers(slot):
        return (
            pltpu.make_async_copy(
                x_hbm.at[idx_v.at[slot, 0]],
                buf_v.at[slot, :, pl.ds(0, _D)], gsems[slot]),
            pltpu.make_async_copy(
                x_hbm.at[idx_v.at[slot, 1]],
                buf_v.at[slot, :, pl.ds(_D, _D)], gsems[slot]),
        )

    def write(k, slot, valid):
        t2_0, b = item(k)
        src = buf_v.at[slot] if valid else zbuf_v
        return pltpu.make_async_copy(
            src, out_hbm.at[pl.ds(t2_0, _BLK), b], wsems[slot])

    def start_item(k, slot):
        @pl.when(item_valid(k))
        def _():
            fill_idx(k, slot)
            for g in gathers(slot):
                g.start()

    def finish_item(k, slot):
        valid = item_valid(k)

        @pl.when(valid)
        def _():
            for g in gathers(slot):
                g.wait()
            write(k, slot, True).start()

        @pl.when(jnp.logical_not(valid))
        def _():
            write(k, slot, False).start()

    start_item(0, 0)

    @pl.loop(0, _N, step=2)
    def _pair(g):
        for p in range(2):
            k = g + p
            slot, nslot = p, 1 - p

            def prefetch():
                @pl.when(k >= 1)
                def _drain():
                    write(k - 1, nslot, True).wait()

                start_item(k + 1, nslot)

            if p == 0:
                prefetch()
            else:
                pl.when(k + 1 < _N)(prefetch)

            finish_item(k, slot)

    write(_N - 2, 0, True).wait()
    write(_N - 1, 1, True).wait()


def kernel(x, lens):
    T, B, D = x.shape
    lens_bc = jnp.broadcast_to(lens.astype(jnp.int32)[:, None], (B, _L))
    zeros = jnp.zeros((_BLK, 2 * D), jnp.float32)
    out = _pool_sc(x.reshape(T * B, D), lens_bc, zeros)
    return out, lens // 2


# speedup vs baseline: 4.3584x; 1.0204x over previous
import functools

import jax
import jax.numpy as jnp
from jax import lax
from jax.experimental import pallas as pl
from jax.experimental.pallas import tpu as pltpu
from jax.experimental.pallas import tpu_sc as plsc

_T, _B, _D = 2048, 16, 1024
_T2 = _T // 2
_NC, _NS, _L = 2, 16, 16
_NW = _NC * _NS
_BLK = 16
_NBLK = _T2 // _BLK
_N = _NBLK * _B // _NW


@functools.partial(
    pl.kernel,
    out_type=jax.ShapeDtypeStruct((_T2, _B, 2 * _D), jnp.float32),
    mesh=plsc.VectorSubcoreMesh(core_axis_name="c", subcore_axis_name="s"),
    scratch_types=[
        pltpu.VMEM((_B, _L), jnp.int32),
        pltpu.VMEM((2, 2, _L), jnp.int32),
        pltpu.VMEM((2, _BLK, 2 * _D), jnp.float32),
        pltpu.VMEM((_BLK, 2 * _D), jnp.float32),
        pltpu.SemaphoreType.DMA,
        pltpu.SemaphoreType.DMA,
        pltpu.SemaphoreType.DMA,
        pltpu.SemaphoreType.DMA,
    ],
)
def _pool_sc(x_hbm, lens_bc_hbm, zeros_hbm, out_hbm,
             lens_v, idx_v, buf_v, zbuf_v, gsem0, gsem1, wsem0, wsem1):
    wid = lax.axis_index("s") * _NC + lax.axis_index("c")
    pltpu.sync_copy(lens_bc_hbm, lens_v)
    pltpu.sync_copy(zeros_hbm, zbuf_v)
    i_vec = lax.iota(jnp.int32, _L)
    gsems = (gsem0, gsem1)
    wsems = (wsem0, wsem1)

    def item(k):
        first = wid * _BLK
        second = (_NBLK - 1 - wid) * _BLK
        half = k >> 4
        b = k & (_B - 1)
        return jnp.where(half == 0, first, second), b

    def item_valid(k):
        t2_0, b = item(k)
        newlens = lens_v[b] >> 1
        return t2_0 < newlens[0]

    def fill_idx(k, slot):
        t2_0, b = item(k)
        lens_b = lens_v[b]
        t2_vec = t2_0 + i_vec
        masked = (lens_b >> 1) <= t2_vec
        for j in range(2):
            t_nat = 2 * t2_vec + j
            t_src = jnp.where(masked, jnp.maximum(t_nat, lens_b), t_nat)
            idx_v[slot, j] = t_src * _B + b

    def gathers(slot):
        return (
            pltpu.make_async_copy(
                x_hbm.at[idx_v.at[slot, 0]],
                buf_v.at[slot, :, pl.ds(0, _D)], gsems[slot]),
            pltpu.make_async_copy(
                x_hbm.at[idx_v.at[slot, 1]],
                buf_v.at[slot, :, pl.ds(_D, _D)], gsems[slot]),
        )

    def write(k, slot, valid):
        t2_0, b = item(k)
        src = buf_v.at[slot] if valid else zbuf_v
        return pltpu.make_async_copy(
            src, out_hbm.at[pl.ds(t2_0, _BLK), b], wsems[slot])

    def start_item(k, slot):
        @pl.when(item_valid(k))
        def _():
            fill_idx(k, slot)
            for g in gathers(slot):
                g.start()

    def finish_item(k, slot):
        valid = item_valid(k)

        @pl.when(valid)
        def _():
            for g in gathers(slot):
                g.wait()
            write(k, slot, True).start()

        @pl.when(jnp.logical_not(valid))
        def _():
            write(k, slot, False).start()

    start_item(0, 0)

    @pl.loop(0, _N, step=2)
    def _pair(g):
        for p in range(2):
            k = g + p
            slot, nslot = p, 1 - p

            def prefetch():
                @pl.when(k >= 1)
                def _drain():
                    write(k - 1, nslot, True).wait()

                start_item(k + 1, nslot)

            if p == 0:
                prefetch()
            else:
                pl.when(k + 1 < _N)(prefetch)

            finish_item(k, slot)

    write(_N - 2, 0, True).wait()
    write(_N - 1, 1, True).wait()


def kernel(x, lens):
    T, B, D = x.shape
    lens_bc = jnp.broadcast_to(lens.astype(jnp.int32)[:, None], (B, _L))
    zeros = jnp.zeros((_BLK, 2 * D), jnp.float32)
    out = _pool_sc(x.reshape(T * B, D), lens_bc, zeros)
    return out, lens // 2
